# baseline (speedup 1.0000x reference)
import functools

import jax
import jax.numpy as jnp
from jax import lax
from jax.experimental import pallas as pl
from jax.experimental.pallas import tpu as pltpu
from jax.experimental.pallas import tpu_sc as plsc

N = 10000
E = 320000
G = 16
D_IN = 128
HID = 256
N_CLS = 2

NC = 2
NS = 16
SLAB = HID // NC

ROWS = 624
TAIL_ROWS = N - NS * ROWS

CH = 128

@functools.cache
def _mesh():
    return plsc.VectorSubcoreMesh(core_axis_name="c", subcore_axis_name="s",
                                  num_cores=NC, num_subcores=NS)


def _fill(ref, n, value, dtype):
    for j in range(n // 16):
        ref[pl.ds(j * 16, 16)] = jnp.full((16,), value, dtype)


_DEG_PER_TILE = E // (NC * NS)
_DEG_CHUNKS = _DEG_PER_TILE // CH
_DEG_TAIL = _DEG_PER_TILE - _DEG_CHUNKS * CH


def _sc_deg(dst):
    return _build_sc_deg()(dst)


@functools.cache
def _build_sc_deg():
    return functools.partial(
        pl.kernel,
        out_type=jax.ShapeDtypeStruct((NC * N,), jnp.float32),
        mesh=_mesh(),
        scratch_types=[
            pltpu.VMEM((CH,), jnp.int32),
            pltpu.VMEM((_DEG_TAIL,), jnp.int32),
            pltpu.VMEM((CH,), jnp.float32),
            pltpu.VMEM((_DEG_TAIL,), jnp.float32),
            pltpu.VMEM((ROWS + TAIL_ROWS,), jnp.float32),
            pltpu.VMEM((ROWS + TAIL_ROWS,), jnp.float32),
            pltpu.VMEM_SHARED((N,), jnp.float32),
        ],
    )(_sc_deg_body)


def _sc_deg_body(dst_hbm, pdeg_hbm, idx_v, idxt_v, ones_v, onest_v, zeros_v,
                 stage_v, deg_sh):
    c = lax.axis_index("c")
    s = lax.axis_index("s")
    _fill(ones_v, CH, 1.0, jnp.float32)
    _fill(onest_v, _DEG_TAIL, 1.0, jnp.float32)
    _fill(zeros_v, ROWS + TAIL_ROWS, 0.0, jnp.float32)

    roff = s * ROWS
    pltpu.sync_copy(zeros_v.at[pl.ds(0, ROWS)], deg_sh.at[pl.ds(roff, ROWS)])

    @pl.when(s == NS - 1)
    def _():
        pltpu.sync_copy(zeros_v.at[pl.ds(0, TAIL_ROWS)],
                        deg_sh.at[pl.ds(NS * ROWS, TAIL_ROWS)])

    plsc.subcore_barrier()

    base = (c * NS + s) * _DEG_PER_TILE

    def _chunk(k, carry):
        off = pl.multiple_of(base + k * CH, 8)
        pltpu.sync_copy(dst_hbm.at[pl.ds(off, CH)], idx_v)
        pltpu.sync_copy(ones_v, deg_sh.at[idx_v], add=True)
        return carry

    lax.fori_loop(0, _DEG_CHUNKS, _chunk, 0)
    toff = pl.multiple_of(base + _DEG_CHUNKS * CH, 8)
    pltpu.sync_copy(dst_hbm.at[pl.ds(toff, _DEG_TAIL)], idxt_v)
    pltpu.sync_copy(onest_v, deg_sh.at[idxt_v], add=True)

    plsc.subcore_barrier()

    coff = c * N
    pltpu.sync_copy(deg_sh.at[pl.ds(roff, ROWS)], stage_v.at[pl.ds(0, ROWS)])
    pltpu.sync_copy(stage_v.at[pl.ds(0, ROWS)],
                    pdeg_hbm.at[pl.ds(coff + roff, ROWS)])

    @pl.when(s == NS - 1)
    def _():
        pltpu.sync_copy(deg_sh.at[pl.ds(NS * ROWS, TAIL_ROWS)],
                        stage_v.at[pl.ds(0, TAIL_ROWS)])
        pltpu.sync_copy(stage_v.at[pl.ds(0, TAIL_ROWS)],
                        pdeg_hbm.at[pl.ds(coff + NS * ROWS, TAIL_ROWS)])


_AGG_PER_TILE = E // NS
_AGG_CHUNKS = _AGG_PER_TILE // CH
_AGG_TAIL = _AGG_PER_TILE - _AGG_CHUNKS * CH


def _sc_agg(z_flat, src, dst):
    return _build_sc_agg()(z_flat, src, dst)


@functools.cache
def _build_sc_agg():
    return functools.partial(
        pl.kernel,
        out_type=jax.ShapeDtypeStruct((NC * N, SLAB), jnp.float32),
        mesh=_mesh(),
        scratch_types=(
            [pltpu.VMEM((CH,), jnp.int32)] * 3
            + [pltpu.VMEM((CH,), jnp.int32)] * 6
            + [pltpu.VMEM((CH,), jnp.int32)] * 3
            + [pltpu.VMEM((CH, SLAB), jnp.float32)] * 3
            + [pltpu.VMEM((_AGG_TAIL,), jnp.int32)] * 3
            + [pltpu.VMEM_SHARED((N, SLAB), jnp.float32)]
            + [pltpu.SemaphoreType.DMA] * 13
        ),
    )(_sc_agg_body)


def _sc_agg_body(z_hbm, src_hbm, dst_hbm, out_hbm, *refs):
    IDX_S = refs[0:3]
    IDX_D = refs[3:9]
    IDX_G = refs[9:12]
    ROWSB = refs[12:15]
    idx_st, idx_dt, idx_gt = refs[15:18]
    agg_sh = refs[18]
    ISEM = refs[19:25]
    GSEM = refs[25:28]
    SSEM = refs[28:31]
    sem = refs[31]
    rows0 = ROWSB[0]
    c = lax.axis_index("c")
    s = lax.axis_index("s")
    coff = c * N

    roff = s * ROWS

    def _move_rows(src_at, dst_at):
        for o, sz in ((0, CH), (CH, CH), (2 * CH, CH), (3 * CH, CH),
                      (4 * CH, ROWS - 4 * CH)):
            stg = rows0.at[pl.ds(0, sz)]
            pltpu.sync_copy(src_at(o, sz), stg)
            pltpu.sync_copy(stg, dst_at(o, sz))

    _move_rows(lambda o, sz: z_hbm.at[pl.ds(coff + roff + o, sz)],
               lambda o, sz: agg_sh.at[pl.ds(roff + o, sz)])

    @pl.when(s == NS - 1)
    def _():
        stg = rows0.at[pl.ds(0, TAIL_ROWS)]
        pltpu.sync_copy(z_hbm.at[pl.ds(coff + NS * ROWS, TAIL_ROWS)], stg)
        pltpu.sync_copy(stg, agg_sh.at[pl.ds(NS * ROWS, TAIL_ROWS)])

    plsc.subcore_barrier()

    base = s * _AGG_PER_TILE

    def _start_idx(k, m6):
        off = pl.multiple_of(base + k * CH, 8)
        pltpu.async_copy(src_hbm.at[pl.ds(off, CH)], IDX_S[m6 % 3], ISEM[m6])
        pltpu.async_copy(dst_hbm.at[pl.ds(off, CH)], IDX_D[m6], ISEM[m6])

    def _launch(m6):
        b = m6 % 3
        pltpu.make_async_copy(src_hbm.at[pl.ds(0, CH)], IDX_S[b],
                              ISEM[m6]).wait()
        pltpu.make_async_copy(dst_hbm.at[pl.ds(0, CH)], IDX_D[m6],
                              ISEM[m6]).wait()
        for j in range(CH // 16):
            sl = pl.ds(j * 16, 16)
            IDX_G[b][sl] = IDX_S[b][sl] + coff
        pltpu.async_copy(z_hbm.at[IDX_G[b]], ROWSB[b], GSEM[b])

    def _drain(b):
        pltpu.make_async_copy(z_hbm.at[pl.ds(0, CH)], ROWSB[b],
                              SSEM[b]).wait()

    def _step(k, m6, drain=True, prefetch=True, nxt=True):
        b = m6 % 3
        pltpu.make_async_copy(z_hbm.at[pl.ds(0, CH)], ROWSB[b],
                              GSEM[b]).wait()
        pltpu.async_copy(ROWSB[b], agg_sh.at[IDX_D[m6]], SSEM[b], add=True)
        if drain:
            _drain((m6 + 1) % 3)
        if prefetch:
            _start_idx(k + 3, (m6 + 3) % 6)
        if nxt:
            _launch((m6 + 1) % 6)

    _start_idx(0, 0)
    _start_idx(1, 1)
    _start_idx(2, 2)
    _launch(0)
    _step(0, 0, drain=False)
    _step(1, 1, drain=False)

    def _block(j, carry):
        k0 = 2 + 12 * j
        for u in range(12):
            _step(k0 + u, (2 + u) % 6)
        return carry

    lax.fori_loop(0, (_AGG_CHUNKS - 12) // 12, _block, 0)

    for k in range(_AGG_CHUNKS - 10, _AGG_CHUNKS):
        _step(k, k % 6, prefetch=k + 3 < _AGG_CHUNKS, nxt=k + 1 < _AGG_CHUNKS)
    _drain((_AGG_CHUNKS - 2) % 3)
    _drain((_AGG_CHUNKS - 1) % 3)

    toff = pl.multiple_of(base + _AGG_CHUNKS * CH, 8)
    pltpu.sync_copy(src_hbm.at[pl.ds(toff, _AGG_TAIL)], idx_st)
    pltpu.sync_copy(dst_hbm.at[pl.ds(toff, _AGG_TAIL)], idx_dt)
    for j in range(_AGG_TAIL // 16):
        sl = pl.ds(j * 16, 16)
        idx_gt[sl] = idx_st[sl] + coff
    rows_t = ROWSB[1].at[pl.ds(0, _AGG_TAIL)]
    pltpu.async_copy(z_hbm.at[idx_gt], rows_t, sem).wait()
    pltpu.sync_copy(rows_t, agg_sh.at[idx_dt], add=True)

    plsc.subcore_barrier()

    _move_rows(lambda o, sz: agg_sh.at[pl.ds(roff + o, sz)],
               lambda o, sz: out_hbm.at[pl.ds(coff + roff + o, sz)])

    @pl.when(s == NS - 1)
    def _():
        stg = rows0.at[pl.ds(0, TAIL_ROWS)]
        pltpu.sync_copy(agg_sh.at[pl.ds(NS * ROWS, TAIL_ROWS)], stg)
        pltpu.sync_copy(stg, out_hbm.at[pl.ds(coff + NS * ROWS, TAIL_ROWS)])


_BLK = 1000
_GRID = N // _BLK


def _dinv_of(p_ref):
    return lax.rsqrt(p_ref[0] + p_ref[1] + 1.0)


def _tc_first_body(x_ref, w_ref, p_ref, o_ref):
    z = _dinv_of(p_ref) * jnp.dot(x_ref[...], w_ref[...],
                                  preferred_element_type=jnp.float32)
    o_ref[0] = z[:, :SLAB]
    o_ref[1] = z[:, SLAB:]


_tc_first = pl.pallas_call(
    _tc_first_body,
    grid=(_GRID,),
    in_specs=[
        pl.BlockSpec((_BLK, D_IN), lambda i: (i, 0)),
        pl.BlockSpec((D_IN, HID), lambda i: (0, 0)),
        pl.BlockSpec((NC, _BLK, 1), lambda i: (0, i, 0)),
    ],
    out_specs=pl.BlockSpec((NC, _BLK, SLAB), lambda i: (0, i, 0)),
    out_shape=jax.ShapeDtypeStruct((NC, N, SLAB), jnp.float32),
)


def _tc_mid_body(a_ref, p_ref, w_ref, b_ref, o_ref):
    dinv = _dinv_of(p_ref)
    h = jnp.concatenate([a_ref[0], a_ref[1]], axis=1)
    h = jnp.maximum(dinv * h + b_ref[...], 0.0)
    z = dinv * jnp.dot(h, w_ref[...], preferred_element_type=jnp.float32)
    o_ref[0] = z[:, :SLAB]
    o_ref[1] = z[:, SLAB:]


_tc_mid = pl.pallas_call(
    _tc_mid_body,
    grid=(_GRID,),
    in_specs=[
        pl.BlockSpec((NC, _BLK, SLAB), lambda i: (0, i, 0)),
        pl.BlockSpec((NC, _BLK, 1), lambda i: (0, i, 0)),
        pl.BlockSpec((HID, HID), lambda i: (0, 0)),
        pl.BlockSpec((1, HID), lambda i: (0, 0)),
    ],
    out_specs=pl.BlockSpec((NC, _BLK, SLAB), lambda i: (0, i, 0)),
    out_shape=jax.ShapeDtypeStruct((NC, N, SLAB), jnp.float32),
)


def _tc_final_body(a_ref, p_ref, b_ref, bt_ref, wl_ref, bl_ref, o_ref,
                   sums, cnt):
    i = pl.program_id(0)

    @pl.when(i == 0)
    def _():
        sums[...] = jnp.zeros_like(sums)
        cnt[...] = jnp.zeros_like(cnt)

    dinv = _dinv_of(p_ref)
    h = jnp.concatenate([a_ref[0], a_ref[1]], axis=1)
    h = jnp.maximum(dinv * h + b_ref[...], 0.0)
    bt = bt_ref[0]
    onehot = (lax.broadcasted_iota(jnp.int32, (G, _BLK), 0) == bt
              ).astype(jnp.float32)
    sums[...] += jnp.dot(onehot, h, preferred_element_type=jnp.float32)
    cnt[...] += jnp.sum(onehot, axis=1, keepdims=True)

    @pl.when(i == _GRID - 1)
    def _():
        pooled = sums[...] / jnp.maximum(cnt[...], 1.0)
        logits = jnp.dot(pooled, wl_ref[...],
                         preferred_element_type=jnp.float32) + bl_ref[...]
        o_ref[...] = jax.nn.sigmoid(logits)


_tc_final = pl.pallas_call(
    _tc_final_body,
    grid=(_GRID,),
    in_specs=[
        pl.BlockSpec((NC, _BLK, SLAB), lambda i: (0, i, 0)),
        pl.BlockSpec((NC, _BLK, 1), lambda i: (0, i, 0)),
        pl.BlockSpec((1, HID), lambda i: (0, 0)),
        pl.BlockSpec((1, 1, _BLK), lambda i: (i, 0, 0)),
        pl.BlockSpec((HID, N_CLS), lambda i: (0, 0)),
        pl.BlockSpec((1, N_CLS), lambda i: (0, 0)),
    ],
    out_specs=pl.BlockSpec((G, N_CLS), lambda i: (0, 0)),
    out_shape=jax.ShapeDtypeStruct((G, N_CLS), jnp.float32),
    scratch_shapes=[
        pltpu.VMEM((G, HID), jnp.float32),
        pltpu.VMEM((G, 1), jnp.float32),
    ],
)


def kernel(x, edge_index, batch, W1, b1, W2, b2, W3, b3, W4, b4, W5, b5,
           Wl, bl):
    src = edge_index[0].astype(jnp.int32)
    dst = edge_index[1].astype(jnp.int32)
    batch3 = batch.astype(jnp.int32).reshape(_GRID, 1, _BLK)

    pdeg3 = _sc_deg(dst).reshape(NC, N, 1)

    z = _tc_first(x, W1, pdeg3)
    agg = _sc_agg(z.reshape(NC * N, SLAB), src, dst)
    for W, b_prev in ((W2, b1), (W3, b2), (W4, b3), (W5, b4)):
        z = _tc_mid(agg.reshape(NC, N, SLAB), pdeg3, W, b_prev.reshape(1, HID))
        agg = _sc_agg(z.reshape(NC * N, SLAB), src, dst)

    return _tc_final(agg.reshape(NC, N, SLAB), pdeg3, b5.reshape(1, HID),
                     batch3, Wl, bl.reshape(1, N_CLS))


# speedup vs baseline: 1.0901x; 1.0901x over previous
import functools

import jax
import jax.numpy as jnp
from jax import lax
from jax.experimental import pallas as pl
from jax.experimental.pallas import tpu as pltpu
from jax.experimental.pallas import tpu_sc as plsc

N = 10000
E = 320000
G = 16
D_IN = 128
HID = 256
N_CLS = 2

NC = 2
NS = 16
SLAB = HID // NC

ROWS = 624
TAIL_ROWS = N - NS * ROWS

CH = 128
PADN = 10240

@functools.cache
def _mesh():
    return plsc.VectorSubcoreMesh(core_axis_name="c", subcore_axis_name="s",
                                  num_cores=NC, num_subcores=NS)


def _fill(ref, n, value, dtype):
    for j in range(n // 16):
        ref[pl.ds(j * 16, 16)] = jnp.full((16,), value, dtype)


_DEG_PER_TILE = E // (NC * NS)
_DEG_CHUNKS = _DEG_PER_TILE // CH
_DEG_TAIL = _DEG_PER_TILE - _DEG_CHUNKS * CH


def _sc_deg(dst):
    return _build_sc_deg()(dst)


@functools.cache
def _build_sc_deg():
    return functools.partial(
        pl.kernel,
        out_type=jax.ShapeDtypeStruct((NC * PADN,), jnp.float32),
        mesh=_mesh(),
        scratch_types=[
            pltpu.VMEM((CH,), jnp.int32),
            pltpu.VMEM((CH,), jnp.int32),
            pltpu.VMEM((_DEG_TAIL,), jnp.int32),
            pltpu.VMEM((CH,), jnp.float32),
            pltpu.VMEM((_DEG_TAIL,), jnp.float32),
            pltpu.VMEM((ROWS + TAIL_ROWS,), jnp.float32),
            pltpu.VMEM((ROWS + TAIL_ROWS,), jnp.float32),
            pltpu.VMEM_SHARED((N,), jnp.float32),
            pltpu.SemaphoreType.DMA,
            pltpu.SemaphoreType.DMA,
        ],
    )(_sc_deg_body)


def _sc_deg_body(dst_hbm, pdeg_hbm, idx0_v, idx1_v, idxt_v, ones_v, onest_v,
                 zeros_v, stage_v, deg_sh, isem0, isem1):
    IDX, ISEM = (idx0_v, idx1_v), (isem0, isem1)
    c = lax.axis_index("c")
    s = lax.axis_index("s")
    _fill(ones_v, CH, 1.0, jnp.float32)
    _fill(onest_v, _DEG_TAIL, 1.0, jnp.float32)
    _fill(zeros_v, ROWS + TAIL_ROWS, 0.0, jnp.float32)

    roff = s * ROWS
    pltpu.sync_copy(zeros_v.at[pl.ds(0, ROWS)], deg_sh.at[pl.ds(roff, ROWS)])

    @pl.when(s == NS - 1)
    def _():
        pltpu.sync_copy(zeros_v.at[pl.ds(0, TAIL_ROWS)],
                        deg_sh.at[pl.ds(NS * ROWS, TAIL_ROWS)])

    plsc.subcore_barrier()

    base = (c * NS + s) * _DEG_PER_TILE

    def _dstart(b, k):
        off = pl.multiple_of(base + k * CH, 8)
        pltpu.async_copy(dst_hbm.at[pl.ds(off, CH)], IDX[b], ISEM[b])

    def _dfin(b):
        pltpu.make_async_copy(dst_hbm.at[pl.ds(0, CH)], IDX[b],
                              ISEM[b]).wait()
        pltpu.sync_copy(ones_v, deg_sh.at[IDX[b]], add=True)

    _dstart(0, 0)
    _dstart(1, 1)

    def _dpair(j, carry):
        k0 = 2 * j
        _dfin(0)
        _dstart(0, k0 + 2)
        _dfin(1)
        _dstart(1, k0 + 3)
        return carry

    lax.fori_loop(0, _DEG_CHUNKS // 2 - 1, _dpair, 0)
    _dfin(0)
    _dfin(1)
    toff = pl.multiple_of(base + _DEG_CHUNKS * CH, 8)
    pltpu.sync_copy(dst_hbm.at[pl.ds(toff, _DEG_TAIL)], idxt_v)
    pltpu.sync_copy(onest_v, deg_sh.at[idxt_v], add=True)

    plsc.subcore_barrier()

    coff = c * PADN
    pltpu.sync_copy(deg_sh.at[pl.ds(roff, ROWS)], stage_v.at[pl.ds(0, ROWS)])
    pltpu.sync_copy(stage_v.at[pl.ds(0, ROWS)],
                    pdeg_hbm.at[pl.ds(coff + roff, ROWS)])

    @pl.when(s == NS - 1)
    def _():
        pltpu.sync_copy(deg_sh.at[pl.ds(NS * ROWS, TAIL_ROWS)],
                        stage_v.at[pl.ds(0, TAIL_ROWS)])
        pltpu.sync_copy(stage_v.at[pl.ds(0, TAIL_ROWS)],
                        pdeg_hbm.at[pl.ds(coff + NS * ROWS, TAIL_ROWS)])


_AGG_PER_TILE = E // NS
_AGG_CHUNKS = _AGG_PER_TILE // CH
_AGG_TAIL = _AGG_PER_TILE - _AGG_CHUNKS * CH


def _sc_agg(z_flat, src, dst):
    return _build_sc_agg()(z_flat, src, dst)


@functools.cache
def _build_sc_agg():
    return functools.partial(
        pl.kernel,
        out_type=jax.ShapeDtypeStruct((NC * N, SLAB), jnp.float32),
        mesh=_mesh(),
        scratch_types=[
            pltpu.VMEM((CH,), jnp.int32),
            pltpu.VMEM((CH,), jnp.int32),
            pltpu.VMEM((CH,), jnp.int32),
            pltpu.VMEM((CH,), jnp.int32),
            pltpu.VMEM((CH,), jnp.int32),
            pltpu.VMEM((CH,), jnp.int32),
            pltpu.VMEM((CH, SLAB), jnp.float32),
            pltpu.VMEM((CH, SLAB), jnp.float32),
            pltpu.VMEM((_AGG_TAIL,), jnp.int32),
            pltpu.VMEM((_AGG_TAIL,), jnp.int32),
            pltpu.VMEM((_AGG_TAIL,), jnp.int32),
            pltpu.VMEM_SHARED((N, SLAB), jnp.float32),
            pltpu.SemaphoreType.DMA,
            pltpu.SemaphoreType.DMA,
            pltpu.SemaphoreType.DMA,
            pltpu.SemaphoreType.DMA,
            pltpu.SemaphoreType.DMA,
        ],
    )(_sc_agg_body)


def _sc_agg_body(z_hbm, src_hbm, dst_hbm, out_hbm,
                 idx_s0, idx_s1, idx_d0, idx_d1, idx_g0, idx_g1,
                 rows0, rows1, idx_st, idx_dt, idx_gt,
                 agg_sh, isem0, isem1, gsem0, gsem1, sem):
    IDX_S, IDX_D, IDX_G = (idx_s0, idx_s1), (idx_d0, idx_d1), (idx_g0, idx_g1)
    ROWSB, ISEM, GSEM = (rows0, rows1), (isem0, isem1), (gsem0, gsem1)
    c = lax.axis_index("c")
    s = lax.axis_index("s")
    coff = c * N

    roff = s * ROWS

    def _move_rows(src_at, dst_at):
        for o, sz in ((0, CH), (CH, CH), (2 * CH, CH), (3 * CH, CH),
                      (4 * CH, ROWS - 4 * CH)):
            stg = rows0.at[pl.ds(0, sz)]
            pltpu.sync_copy(src_at(o, sz), stg)
            pltpu.sync_copy(stg, dst_at(o, sz))

    _move_rows(lambda o, sz: z_hbm.at[pl.ds(coff + roff + o, sz)],
               lambda o, sz: agg_sh.at[pl.ds(roff + o, sz)])

    @pl.when(s == NS - 1)
    def _():
        stg = rows0.at[pl.ds(0, TAIL_ROWS)]
        pltpu.sync_copy(z_hbm.at[pl.ds(coff + NS * ROWS, TAIL_ROWS)], stg)
        pltpu.sync_copy(stg, agg_sh.at[pl.ds(NS * ROWS, TAIL_ROWS)])

    plsc.subcore_barrier()

    base = s * _AGG_PER_TILE

    def _start(b, k):
        off = pl.multiple_of(base + k * CH, 8)
        pltpu.async_copy(src_hbm.at[pl.ds(off, CH)], IDX_S[b], ISEM[b])
        pltpu.async_copy(dst_hbm.at[pl.ds(off, CH)], IDX_D[b], ISEM[b])

    def _launch(b):
        pltpu.make_async_copy(src_hbm.at[pl.ds(0, CH)], IDX_S[b],
                              ISEM[b]).wait()
        pltpu.make_async_copy(dst_hbm.at[pl.ds(0, CH)], IDX_D[b],
                              ISEM[b]).wait()
        for j in range(CH // 16):
            sl = pl.ds(j * 16, 16)
            IDX_G[b][sl] = IDX_S[b][sl] + coff
        pltpu.async_copy(z_hbm.at[IDX_G[b]], ROWSB[b], GSEM[b])

    def _finish(b):
        pltpu.make_async_copy(z_hbm.at[pl.ds(0, CH)], ROWSB[b],
                              GSEM[b]).wait()
        pltpu.sync_copy(ROWSB[b], agg_sh.at[IDX_D[b]], add=True)

    _start(0, 0)
    _launch(0)
    _start(1, 1)

    def _pair(j, carry):
        k0 = 2 * j
        _launch(1)
        _finish(0)
        _start(0, k0 + 2)
        _launch(0)
        _finish(1)
        _start(1, k0 + 3)
        return carry

    lax.fori_loop(0, _AGG_CHUNKS // 2 - 1, _pair, 0)
    _launch(1)
    _finish(0)
    _finish(1)

    toff = pl.multiple_of(base + _AGG_CHUNKS * CH, 8)
    pltpu.sync_copy(src_hbm.at[pl.ds(toff, _AGG_TAIL)], idx_st)
    pltpu.sync_copy(dst_hbm.at[pl.ds(toff, _AGG_TAIL)], idx_dt)
    for j in range(_AGG_TAIL // 16):
        sl = pl.ds(j * 16, 16)
        idx_gt[sl] = idx_st[sl] + coff
    rows_t = rows1.at[pl.ds(0, _AGG_TAIL)]
    pltpu.async_copy(z_hbm.at[idx_gt], rows_t, sem).wait()
    pltpu.sync_copy(rows_t, agg_sh.at[idx_dt], add=True)

    plsc.subcore_barrier()

    _move_rows(lambda o, sz: agg_sh.at[pl.ds(roff + o, sz)],
               lambda o, sz: out_hbm.at[pl.ds(coff + roff + o, sz)])

    @pl.when(s == NS - 1)
    def _():
        stg = rows0.at[pl.ds(0, TAIL_ROWS)]
        pltpu.sync_copy(agg_sh.at[pl.ds(NS * ROWS, TAIL_ROWS)], stg)
        pltpu.sync_copy(stg, out_hbm.at[pl.ds(coff + NS * ROWS, TAIL_ROWS)])


_BLK = 1000
_GRID = N // _BLK


_DBLK = 1024


def _tc_dinv_body(p0_ref, p1_ref, o_ref):
    deg = p0_ref[...] + p1_ref[...] + 1.0
    o_ref[...] = lax.rsqrt(deg).reshape(_DBLK, 1)


_tc_dinv = pl.pallas_call(
    _tc_dinv_body,
    grid=(PADN // _DBLK,),
    in_specs=[
        pl.BlockSpec((_DBLK,), lambda i: (i,)),
        pl.BlockSpec((_DBLK,), lambda i: (i + PADN // _DBLK,)),
    ],
    out_specs=pl.BlockSpec((_DBLK, 1), lambda i: (i, 0)),
    out_shape=jax.ShapeDtypeStruct((PADN, 1), jnp.float32),
)


_EBLK = 2000


def _tc_split_body(e_ref, os_ref, od_ref):
    os_ref[...] = e_ref[0]
    od_ref[...] = e_ref[1]


_tc_split = pl.pallas_call(
    _tc_split_body,
    out_shape=[
        jax.ShapeDtypeStruct((E,), jnp.int32),
        jax.ShapeDtypeStruct((E,), jnp.int32),
    ],
)


def _tc_first_body(x_ref, w_ref, v_ref, o_ref):
    z = v_ref[...] * jnp.dot(x_ref[...], w_ref[...],
                             preferred_element_type=jnp.float32)
    o_ref[0] = z[:, :SLAB]
    o_ref[1] = z[:, SLAB:]


_tc_first = pl.pallas_call(
    _tc_first_body,
    grid=(_GRID,),
    in_specs=[
        pl.BlockSpec((_BLK, D_IN), lambda i: (i, 0)),
        pl.BlockSpec((D_IN, HID), lambda i: (0, 0)),
        pl.BlockSpec((_BLK, 1), lambda i: (i, 0)),
    ],
    out_specs=pl.BlockSpec((NC, _BLK, SLAB), lambda i: (0, i, 0)),
    out_shape=jax.ShapeDtypeStruct((NC, N, SLAB), jnp.float32),
)


def _tc_mid_body(a_ref, v_ref, w_ref, b_ref, o_ref):
    dinv = v_ref[...]
    h = jnp.concatenate([a_ref[0], a_ref[1]], axis=1)
    h = jnp.maximum(dinv * h + b_ref[...], 0.0)
    z = dinv * jnp.dot(h, w_ref[...], preferred_element_type=jnp.float32)
    o_ref[0] = z[:, :SLAB]
    o_ref[1] = z[:, SLAB:]


_tc_mid = pl.pallas_call(
    _tc_mid_body,
    grid=(_GRID,),
    in_specs=[
        pl.BlockSpec((NC, _BLK, SLAB), lambda i: (0, i, 0)),
        pl.BlockSpec((_BLK, 1), lambda i: (i, 0)),
        pl.BlockSpec((HID, HID), lambda i: (0, 0)),
        pl.BlockSpec((1, HID), lambda i: (0, 0)),
    ],
    out_specs=pl.BlockSpec((NC, _BLK, SLAB), lambda i: (0, i, 0)),
    out_shape=jax.ShapeDtypeStruct((NC, N, SLAB), jnp.float32),
)


def _tc_final_body(a_ref, v_ref, b_ref, bt_ref, wl_ref, bl_ref, o_ref,
                   sums, cnt):
    i = pl.program_id(0)

    @pl.when(i == 0)
    def _():
        sums[...] = jnp.zeros_like(sums)
        cnt[...] = jnp.zeros_like(cnt)

    dinv = v_ref[...]
    h = jnp.concatenate([a_ref[0], a_ref[1]], axis=1)
    h = jnp.maximum(dinv * h + b_ref[...], 0.0)
    bt = bt_ref[0]
    onehot = (lax.broadcasted_iota(jnp.int32, (G, _BLK), 0) == bt
              ).astype(jnp.float32)
    sums[...] += jnp.dot(onehot, h, preferred_element_type=jnp.float32)
    cnt[...] += jnp.sum(onehot, axis=1, keepdims=True)

    @pl.when(i == _GRID - 1)
    def _():
        pooled = sums[...] / jnp.maximum(cnt[...], 1.0)
        logits = jnp.dot(pooled, wl_ref[...],
                         preferred_element_type=jnp.float32) + bl_ref[...]
        o_ref[...] = jax.nn.sigmoid(logits)


_tc_final = pl.pallas_call(
    _tc_final_body,
    grid=(_GRID,),
    in_specs=[
        pl.BlockSpec((NC, _BLK, SLAB), lambda i: (0, i, 0)),
        pl.BlockSpec((_BLK, 1), lambda i: (i, 0)),
        pl.BlockSpec((1, HID), lambda i: (0, 0)),
        pl.BlockSpec((1, 1, _BLK), lambda i: (i, 0, 0)),
        pl.BlockSpec((HID, N_CLS), lambda i: (0, 0)),
        pl.BlockSpec((1, N_CLS), lambda i: (0, 0)),
    ],
    out_specs=pl.BlockSpec((G, N_CLS), lambda i: (0, 0)),
    out_shape=jax.ShapeDtypeStruct((G, N_CLS), jnp.float32),
    scratch_shapes=[
        pltpu.VMEM((G, HID), jnp.float32),
        pltpu.VMEM((G, 1), jnp.float32),
    ],
)


def kernel(x, edge_index, batch, W1, b1, W2, b2, W3, b3, W4, b4, W5, b5,
           Wl, bl):
    src, dst = _tc_split(edge_index.astype(jnp.int32))
    batch3 = batch.astype(jnp.int32).reshape(_GRID, 1, _BLK)

    dinv = _tc_dinv(*((_sc_deg(dst),) * 2))

    z = _tc_first(x, W1, dinv)
    agg = _sc_agg(z.reshape(NC * N, SLAB), src, dst)
    for W, b_prev in ((W2, b1), (W3, b2), (W4, b3), (W5, b4)):
        z = _tc_mid(agg.reshape(NC, N, SLAB), dinv, W, b_prev.reshape(1, HID))
        agg = _sc_agg(z.reshape(NC * N, SLAB), src, dst)

    return _tc_final(agg.reshape(NC, N, SLAB), dinv, b5.reshape(1, HID),
                     batch3, Wl, bl.reshape(1, N_CLS))


# speedup vs baseline: 1.0934x; 1.0030x over previous
import functools

import jax
import jax.numpy as jnp
from jax import lax
from jax.experimental import pallas as pl
from jax.experimental.pallas import tpu as pltpu
from jax.experimental.pallas import tpu_sc as plsc

N = 10000
E = 320000
G = 16
D_IN = 128
HID = 256
N_CLS = 2

NC = 2
NS = 16
SLAB = HID // NC

ROWS = 624
TAIL_ROWS = N - NS * ROWS

CH = 128
PADN = 10240

@functools.cache
def _mesh():
    return plsc.VectorSubcoreMesh(core_axis_name="c", subcore_axis_name="s",
                                  num_cores=NC, num_subcores=NS)


def _fill(ref, n, value, dtype):
    for j in range(n // 16):
        ref[pl.ds(j * 16, 16)] = jnp.full((16,), value, dtype)


_DEG_PER_TILE = E // (NC * NS)
_DEG_CHUNKS = _DEG_PER_TILE // CH
_DEG_TAIL = _DEG_PER_TILE - _DEG_CHUNKS * CH


def _sc_deg(dst):
    return _build_sc_deg()(dst)


@functools.cache
def _build_sc_deg():
    return functools.partial(
        pl.kernel,
        out_type=jax.ShapeDtypeStruct((NC * PADN,), jnp.float32),
        mesh=_mesh(),
        scratch_types=[
            pltpu.VMEM((CH,), jnp.int32),
            pltpu.VMEM((CH,), jnp.int32),
            pltpu.VMEM((_DEG_TAIL,), jnp.int32),
            pltpu.VMEM((CH,), jnp.float32),
            pltpu.VMEM((_DEG_TAIL,), jnp.float32),
            pltpu.VMEM((ROWS + TAIL_ROWS,), jnp.float32),
            pltpu.VMEM((ROWS + TAIL_ROWS,), jnp.float32),
            pltpu.VMEM_SHARED((N,), jnp.float32),
            pltpu.SemaphoreType.DMA,
            pltpu.SemaphoreType.DMA,
        ],
    )(_sc_deg_body)


def _sc_deg_body(dst_hbm, pdeg_hbm, idx0_v, idx1_v, idxt_v, ones_v, onest_v,
                 zeros_v, stage_v, deg_sh, isem0, isem1):
    IDX, ISEM = (idx0_v, idx1_v), (isem0, isem1)
    c = lax.axis_index("c")
    s = lax.axis_index("s")
    _fill(ones_v, CH, 1.0, jnp.float32)
    _fill(onest_v, _DEG_TAIL, 1.0, jnp.float32)
    _fill(zeros_v, ROWS + TAIL_ROWS, 0.0, jnp.float32)

    roff = s * ROWS
    pltpu.sync_copy(zeros_v.at[pl.ds(0, ROWS)], deg_sh.at[pl.ds(roff, ROWS)])

    @pl.when(s == NS - 1)
    def _():
        pltpu.sync_copy(zeros_v.at[pl.ds(0, TAIL_ROWS)],
                        deg_sh.at[pl.ds(NS * ROWS, TAIL_ROWS)])

    plsc.subcore_barrier()

    base = (c * NS + s) * _DEG_PER_TILE

    def _dstart(b, k):
        off = pl.multiple_of(base + k * CH, 8)
        pltpu.async_copy(dst_hbm.at[pl.ds(off, CH)], IDX[b], ISEM[b])

    def _dfin(b):
        pltpu.make_async_copy(dst_hbm.at[pl.ds(0, CH)], IDX[b],
                              ISEM[b]).wait()
        pltpu.sync_copy(ones_v, deg_sh.at[IDX[b]], add=True)

    _dstart(0, 0)
    _dstart(1, 1)

    def _dpair(j, carry):
        k0 = 2 * j
        _dfin(0)
        _dstart(0, k0 + 2)
        _dfin(1)
        _dstart(1, k0 + 3)
        return carry

    lax.fori_loop(0, _DEG_CHUNKS // 2 - 1, _dpair, 0)
    _dfin(0)
    _dfin(1)
    toff = pl.multiple_of(base + _DEG_CHUNKS * CH, 8)
    pltpu.sync_copy(dst_hbm.at[pl.ds(toff, _DEG_TAIL)], idxt_v)
    pltpu.sync_copy(onest_v, deg_sh.at[idxt_v], add=True)

    plsc.subcore_barrier()

    coff = c * PADN
    pltpu.sync_copy(deg_sh.at[pl.ds(roff, ROWS)], stage_v.at[pl.ds(0, ROWS)])
    pltpu.sync_copy(stage_v.at[pl.ds(0, ROWS)],
                    pdeg_hbm.at[pl.ds(coff + roff, ROWS)])

    @pl.when(s == NS - 1)
    def _():
        pltpu.sync_copy(deg_sh.at[pl.ds(NS * ROWS, TAIL_ROWS)],
                        stage_v.at[pl.ds(0, TAIL_ROWS)])
        pltpu.sync_copy(stage_v.at[pl.ds(0, TAIL_ROWS)],
                        pdeg_hbm.at[pl.ds(coff + NS * ROWS, TAIL_ROWS)])


_AGG_PER_TILE = E // NS
_AGG_CHUNKS = _AGG_PER_TILE // CH
_AGG_TAIL = _AGG_PER_TILE - _AGG_CHUNKS * CH


def _sc_agg(z_flat, src, dst):
    return _build_sc_agg()(z_flat, src, dst)


@functools.cache
def _build_sc_agg():
    return functools.partial(
        pl.kernel,
        out_type=jax.ShapeDtypeStruct((NC * N, SLAB), jnp.float32),
        mesh=_mesh(),
        scratch_types=[
            pltpu.VMEM((CH,), jnp.int32),
            pltpu.VMEM((CH,), jnp.int32),
            pltpu.VMEM((CH,), jnp.int32),
            pltpu.VMEM((CH,), jnp.int32),
            pltpu.VMEM((CH, SLAB), jnp.float32),
            pltpu.VMEM((CH, SLAB), jnp.float32),
            pltpu.VMEM((_AGG_TAIL,), jnp.int32),
            pltpu.VMEM((_AGG_TAIL,), jnp.int32),
            pltpu.VMEM_SHARED((N, SLAB), jnp.float32),
            pltpu.SemaphoreType.DMA,
            pltpu.SemaphoreType.DMA,
            pltpu.SemaphoreType.DMA,
            pltpu.SemaphoreType.DMA,
            pltpu.SemaphoreType.DMA,
        ],
    )(_sc_agg_body)


def _sc_agg_body(z_hbm, srcx_hbm, dst_hbm, out_hbm,
                 idx_s0, idx_s1, idx_d0, idx_d1,
                 rows0, rows1, idx_st, idx_dt,
                 agg_sh, isem0, isem1, gsem0, gsem1, sem):
    IDX_S, IDX_D = (idx_s0, idx_s1), (idx_d0, idx_d1)
    ROWSB, ISEM, GSEM = (rows0, rows1), (isem0, isem1), (gsem0, gsem1)
    c = lax.axis_index("c")
    s = lax.axis_index("s")
    coff = c * N
    ceoff = c * E

    roff = s * ROWS

    def _move_rows(src_at, dst_at):
        for o, sz in ((0, CH), (CH, CH), (2 * CH, CH), (3 * CH, CH),
                      (4 * CH, ROWS - 4 * CH)):
            stg = rows0.at[pl.ds(0, sz)]
            pltpu.sync_copy(src_at(o, sz), stg)
            pltpu.sync_copy(stg, dst_at(o, sz))

    _move_rows(lambda o, sz: z_hbm.at[pl.ds(coff + roff + o, sz)],
               lambda o, sz: agg_sh.at[pl.ds(roff + o, sz)])

    @pl.when(s == NS - 1)
    def _():
        stg = rows0.at[pl.ds(0, TAIL_ROWS)]
        pltpu.sync_copy(z_hbm.at[pl.ds(coff + NS * ROWS, TAIL_ROWS)], stg)
        pltpu.sync_copy(stg, agg_sh.at[pl.ds(NS * ROWS, TAIL_ROWS)])

    plsc.subcore_barrier()

    base = s * _AGG_PER_TILE

    def _start(b, k):
        off = pl.multiple_of(base + k * CH, 8)
        pltpu.async_copy(srcx_hbm.at[pl.ds(ceoff + off, CH)], IDX_S[b],
                         ISEM[b])
        pltpu.async_copy(dst_hbm.at[pl.ds(off, CH)], IDX_D[b], ISEM[b])

    def _launch(b):
        pltpu.make_async_copy(srcx_hbm.at[pl.ds(0, CH)], IDX_S[b],
                              ISEM[b]).wait()
        pltpu.make_async_copy(dst_hbm.at[pl.ds(0, CH)], IDX_D[b],
                              ISEM[b]).wait()
        pltpu.async_copy(z_hbm.at[IDX_S[b]], ROWSB[b], GSEM[b])

    def _finish(b):
        pltpu.make_async_copy(z_hbm.at[pl.ds(0, CH)], ROWSB[b],
                              GSEM[b]).wait()
        pltpu.sync_copy(ROWSB[b], agg_sh.at[IDX_D[b]], add=True)

    _start(0, 0)
    _launch(0)
    _start(1, 1)

    def _pair(j, carry):
        k0 = 2 * j
        _launch(1)
        _finish(0)
        _start(0, k0 + 2)
        _launch(0)
        _finish(1)
        _start(1, k0 + 3)
        return carry

    lax.fori_loop(0, _AGG_CHUNKS // 2 - 1, _pair, 0)
    _launch(1)
    _finish(0)
    _finish(1)

    toff = pl.multiple_of(base + _AGG_CHUNKS * CH, 8)
    pltpu.sync_copy(srcx_hbm.at[pl.ds(ceoff + toff, _AGG_TAIL)], idx_st)
    pltpu.sync_copy(dst_hbm.at[pl.ds(toff, _AGG_TAIL)], idx_dt)
    rows_t = rows1.at[pl.ds(0, _AGG_TAIL)]
    pltpu.async_copy(z_hbm.at[idx_st], rows_t, sem).wait()
    pltpu.sync_copy(rows_t, agg_sh.at[idx_dt], add=True)

    plsc.subcore_barrier()

    _move_rows(lambda o, sz: agg_sh.at[pl.ds(roff + o, sz)],
               lambda o, sz: out_hbm.at[pl.ds(coff + roff + o, sz)])

    @pl.when(s == NS - 1)
    def _():
        stg = rows0.at[pl.ds(0, TAIL_ROWS)]
        pltpu.sync_copy(agg_sh.at[pl.ds(NS * ROWS, TAIL_ROWS)], stg)
        pltpu.sync_copy(stg, out_hbm.at[pl.ds(coff + NS * ROWS, TAIL_ROWS)])


_BLK = 1000
_GRID = N // _BLK


_DBLK = 1024


def _tc_dinv_body(p0_ref, p1_ref, o_ref):
    deg = p0_ref[...] + p1_ref[...] + 1.0
    o_ref[...] = lax.rsqrt(deg).reshape(_DBLK, 1)


_tc_dinv = pl.pallas_call(
    _tc_dinv_body,
    grid=(PADN // _DBLK,),
    in_specs=[
        pl.BlockSpec((_DBLK,), lambda i: (i,)),
        pl.BlockSpec((_DBLK,), lambda i: (i + PADN // _DBLK,)),
    ],
    out_specs=pl.BlockSpec((_DBLK, 1), lambda i: (i, 0)),
    out_shape=jax.ShapeDtypeStruct((PADN, 1), jnp.float32),
)


_EBLK = 2000


def _tc_split_body(e_ref, os_ref, od_ref):
    os_ref[pl.ds(0, E)] = e_ref[0]
    os_ref[pl.ds(E, E)] = e_ref[0] + N
    od_ref[...] = e_ref[1]


_tc_split = pl.pallas_call(
    _tc_split_body,
    out_shape=[
        jax.ShapeDtypeStruct((2 * E,), jnp.int32),
        jax.ShapeDtypeStruct((E,), jnp.int32),
    ],
)


def _tc_first_body(x_ref, w_ref, v_ref, o_ref):
    z = v_ref[...] * jnp.dot(x_ref[...], w_ref[...],
                             preferred_element_type=jnp.float32)
    o_ref[0] = z[:, :SLAB]
    o_ref[1] = z[:, SLAB:]


_tc_first = pl.pallas_call(
    _tc_first_body,
    grid=(_GRID,),
    in_specs=[
        pl.BlockSpec((_BLK, D_IN), lambda i: (i, 0)),
        pl.BlockSpec((D_IN, HID), lambda i: (0, 0)),
        pl.BlockSpec((_BLK, 1), lambda i: (i, 0)),
    ],
    out_specs=pl.BlockSpec((NC, _BLK, SLAB), lambda i: (0, i, 0)),
    out_shape=jax.ShapeDtypeStruct((NC, N, SLAB), jnp.float32),
)


def _tc_mid_body(a_ref, v_ref, w_ref, b_ref, o_ref):
    dinv = v_ref[...]
    h = jnp.concatenate([a_ref[0], a_ref[1]], axis=1)
    h = jnp.maximum(dinv * h + b_ref[...], 0.0)
    z = dinv * jnp.dot(h, w_ref[...], preferred_element_type=jnp.float32)
    o_ref[0] = z[:, :SLAB]
    o_ref[1] = z[:, SLAB:]


_tc_mid = pl.pallas_call(
    _tc_mid_body,
    grid=(_GRID,),
    in_specs=[
        pl.BlockSpec((NC, _BLK, SLAB), lambda i: (0, i, 0)),
        pl.BlockSpec((_BLK, 1), lambda i: (i, 0)),
        pl.BlockSpec((HID, HID), lambda i: (0, 0)),
        pl.BlockSpec((1, HID), lambda i: (0, 0)),
    ],
    out_specs=pl.BlockSpec((NC, _BLK, SLAB), lambda i: (0, i, 0)),
    out_shape=jax.ShapeDtypeStruct((NC, N, SLAB), jnp.float32),
)


def _tc_final_body(a_ref, v_ref, b_ref, bt_ref, wl_ref, bl_ref, o_ref,
                   sums, cnt):
    i = pl.program_id(0)

    @pl.when(i == 0)
    def _():
        sums[...] = jnp.zeros_like(sums)
        cnt[...] = jnp.zeros_like(cnt)

    dinv = v_ref[...]
    h = jnp.concatenate([a_ref[0], a_ref[1]], axis=1)
    h = jnp.maximum(dinv * h + b_ref[...], 0.0)
    bt = bt_ref[0]
    onehot = (lax.broadcasted_iota(jnp.int32, (G, _BLK), 0) == bt
              ).astype(jnp.float32)
    sums[...] += jnp.dot(onehot, h, preferred_element_type=jnp.float32)
    cnt[...] += jnp.sum(onehot, axis=1, keepdims=True)

    @pl.when(i == _GRID - 1)
    def _():
        pooled = sums[...] / jnp.maximum(cnt[...], 1.0)
        logits = jnp.dot(pooled, wl_ref[...],
                         preferred_element_type=jnp.float32) + bl_ref[...]
        o_ref[...] = jax.nn.sigmoid(logits)


_tc_final = pl.pallas_call(
    _tc_final_body,
    grid=(_GRID,),
    in_specs=[
        pl.BlockSpec((NC, _BLK, SLAB), lambda i: (0, i, 0)),
        pl.BlockSpec((_BLK, 1), lambda i: (i, 0)),
        pl.BlockSpec((1, HID), lambda i: (0, 0)),
        pl.BlockSpec((1, 1, _BLK), lambda i: (i, 0, 0)),
        pl.BlockSpec((HID, N_CLS), lambda i: (0, 0)),
        pl.BlockSpec((1, N_CLS), lambda i: (0, 0)),
    ],
    out_specs=pl.BlockSpec((G, N_CLS), lambda i: (0, 0)),
    out_shape=jax.ShapeDtypeStruct((G, N_CLS), jnp.float32),
    scratch_shapes=[
        pltpu.VMEM((G, HID), jnp.float32),
        pltpu.VMEM((G, 1), jnp.float32),
    ],
)


def kernel(x, edge_index, batch, W1, b1, W2, b2, W3, b3, W4, b4, W5, b5,
           Wl, bl):
    srcx, dst = _tc_split(edge_index.astype(jnp.int32))
    batch3 = batch.astype(jnp.int32).reshape(_GRID, 1, _BLK)

    dinv = _tc_dinv(*((_sc_deg(dst),) * 2))

    z = _tc_first(x, W1, dinv)
    agg = _sc_agg(z.reshape(NC * N, SLAB), srcx, dst)
    for W, b_prev in ((W2, b1), (W3, b2), (W4, b3), (W5, b4)):
        z = _tc_mid(agg.reshape(NC, N, SLAB), dinv, W, b_prev.reshape(1, HID))
        agg = _sc_agg(z.reshape(NC * N, SLAB), srcx, dst)

    return _tc_final(agg.reshape(NC, N, SLAB), dinv, b5.reshape(1, HID),
                     batch3, Wl, bl.reshape(1, N_CLS))
